# baseline (speedup 1.0000x reference)
import jax
import jax.numpy as jnp
from jax.experimental import pallas as pl
from jax.experimental.pallas import tpu as pltpu

_B, _H, _S, _D, _Q = 8, 16, 4096, 128, 8
_BH = _B * _H
_BHB = 2
_W = 8


def _fused_kernel(pos_ref, kc_ref, vc_ref, kv_ref, vv_ref, ko_ref, vo_ref):
    ko_ref[...] = kc_ref[...]
    vo_ref[...] = vc_ref[...]
    sub_iota = jax.lax.broadcasted_iota(jnp.int32, (1, _W, 1), 1)
    for q in range(_Q):
        pos = pos_ref[q]
        wb = pl.multiple_of((pos // _W) * _W, _W)
        r = pos % _W
        mask = sub_iota == r
        ko_ref[:, pl.ds(wb, _W), :] = jnp.where(
            mask, kv_ref[:, q:q + 1, :], ko_ref[:, pl.ds(wb, _W), :])
        vo_ref[:, pl.ds(wb, _W), :] = jnp.where(
            mask, vv_ref[:, q:q + 1, :], vo_ref[:, pl.ds(wb, _W), :])


def kernel(k_cache, v_cache, input_pos, k_val, v_val):
    kc = k_cache.reshape(_BH, _S, _D)
    vc = v_cache.reshape(_BH, _S, _D)
    kv = k_val.reshape(_BH, _Q, _D)
    vv = v_val.reshape(_BH, _Q, _D)
    grid = (_BH // _BHB,)
    cache_spec = pl.BlockSpec((_BHB, _S, _D), lambda i: (i, 0, 0))
    val_spec = pl.BlockSpec((_BHB, _Q, _D), lambda i: (i, 0, 0))
    ko, vo = pl.pallas_call(
        _fused_kernel,
        grid=grid,
        in_specs=[
            pl.BlockSpec(memory_space=pltpu.SMEM),
            cache_spec, cache_spec, val_spec, val_spec,
        ],
        out_specs=(cache_spec, cache_spec),
        out_shape=(jax.ShapeDtypeStruct((_BH, _S, _D), kc.dtype),
                   jax.ShapeDtypeStruct((_BH, _S, _D), vc.dtype)),
    )(input_pos, kc, vc, kv, vv)
    K = ko.reshape(_B, _H, _S, _D)
    V = vo.reshape(_B, _H, _S, _D)
    return (K, K, V)


# speedup vs baseline: 1.0121x; 1.0121x over previous
import jax
import jax.numpy as jnp
from jax.experimental import pallas as pl
from jax.experimental.pallas import tpu as pltpu

_B, _H, _S, _D, _Q = 8, 16, 4096, 128, 8
_BH = _B * _H
_BHB = 8
_W = 8


def _fused_kernel(pos_ref, c_ref, v_ref, o_ref):
    o_ref[...] = c_ref[...]
    sub_iota = jax.lax.broadcasted_iota(jnp.int32, (1, _W, 1), 1)
    for q in range(_Q):
        pos = pos_ref[q]
        wb = pl.multiple_of((pos // _W) * _W, _W)
        r = pos % _W
        mask = sub_iota == r
        o_ref[:, pl.ds(wb, _W), :] = jnp.where(
            mask, v_ref[:, q:q + 1, :], o_ref[:, pl.ds(wb, _W), :])


def _update(cache, vals, input_pos):
    c = cache.reshape(_BH, _S, _D)
    v = vals.reshape(_BH, _Q, _D)
    grid = (_BH // _BHB,)
    cache_spec = pl.BlockSpec((_BHB, _S, _D), lambda i: (i, 0, 0))
    val_spec = pl.BlockSpec((_BHB, _Q, _D), lambda i: (i, 0, 0))
    out = pl.pallas_call(
        _fused_kernel,
        grid=grid,
        in_specs=[
            pl.BlockSpec(memory_space=pltpu.SMEM),
            cache_spec, val_spec,
        ],
        out_specs=cache_spec,
        out_shape=jax.ShapeDtypeStruct((_BH, _S, _D), c.dtype),
    )(input_pos, c, v)
    return out.reshape(_B, _H, _S, _D)


def kernel(k_cache, v_cache, input_pos, k_val, v_val):
    K = _update(k_cache, k_val, input_pos)
    V = _update(v_cache, v_val, input_pos)
    return (K, K, V)
